# baseline (speedup 1.0000x reference)
import functools

import jax
import jax.numpy as jnp
from jax import lax
from jax.experimental import pallas as pl
from jax.experimental.pallas import tpu as pltpu

_IOU_T = 0.5
_BLK = 256


def _nms_body(bt_ref, brow_ref, st_ref, ot_ref, os_ref,
              s_ref, keep_ref, *, n_pad, blk):
    i = pl.program_id(0)
    j = pl.program_id(1)
    ibase = pl.multiple_of(i * blk, blk)
    jbase = pl.multiple_of(j * blk, blk)

    @pl.when((i == 0) & (j == 0))
    def _():
        keep_ref[...] = jnp.ones_like(keep_ref)

    @pl.when(j >= i)
    def _active():
        x1 = brow_ref[:, 0:1]
        y1 = brow_ref[:, 1:2]
        x2 = brow_ref[:, 2:3]
        y2 = brow_ref[:, 3:4]
        cx1 = bt_ref[0:1, pl.ds(jbase, blk)]
        cy1 = bt_ref[1:2, pl.ds(jbase, blk)]
        cx2 = bt_ref[2:3, pl.ds(jbase, blk)]
        cy2 = bt_ref[3:4, pl.ds(jbase, blk)]
        area_r = (x2 - x1) * (y2 - y1)
        area_c = (cx2 - cx1) * (cy2 - cy1)
        iw = jnp.maximum(jnp.minimum(x2, cx2) - jnp.maximum(x1, cx1), 0.0)
        ih = jnp.maximum(jnp.minimum(y2, cy2) - jnp.maximum(y1, cy1), 0.0)
        inter = iw * ih
        union = jnp.maximum(area_r + area_c - inter, 1e-9)
        s = jnp.where(inter > _IOU_T * union, 1.0, 0.0)

        @pl.when(j == i)
        def _diagonal():
            tri_r = lax.broadcasted_iota(jnp.int32, (blk, blk), 0)
            tri_c = lax.broadcasted_iota(jnp.int32, (blk, blk), 1)
            s_ref[...] = jnp.where(tri_c > tri_r, s, 0.0)

            keep0 = keep_ref[:, pl.ds(ibase, blk)]

            def _cond(carry):
                return carry[1]

            def _sweep(carry):
                kb, _ = carry
                cnt = jnp.dot(kb, s_ref[...],
                              preferred_element_type=jnp.float32)
                kb_new = keep0 * jnp.where(cnt > 0.5, 0.0, 1.0)
                changed = jnp.sum(jnp.abs(kb_new - kb)) > 0.0
                return (kb_new, changed)

            kb, _ = lax.while_loop(_cond, _sweep, (keep0, True))
            keep_ref[:, pl.ds(ibase, blk)] = kb

            ot_ref[...] = bt_ref[:, pl.ds(ibase, blk)] * kb
            os_ref[...] = st_ref[:, pl.ds(ibase, blk)] * kb

        @pl.when(j > i)
        def _tail():
            kb = keep_ref[:, pl.ds(ibase, blk)]
            cnt = jnp.dot(kb, s, preferred_element_type=jnp.float32)
            keep_ref[:, pl.ds(jbase, blk)] = (
                keep_ref[:, pl.ds(jbase, blk)]
                * jnp.where(cnt > 0.5, 0.0, 1.0))


def kernel(boxes, scores):
    n = boxes.shape[0]
    blk = _BLK
    nb = -(-n // blk)
    n_pad = nb * blk

    order = jnp.argsort(-scores)
    b = jnp.take(boxes, order, axis=0)
    s = jnp.take(scores, order, axis=0)
    bp = jnp.concatenate(
        [b, jnp.zeros((n_pad - n, 4), jnp.float32)], axis=0)
    st = jnp.concatenate(
        [s, jnp.zeros((n_pad - n,), jnp.float32)], axis=0)[None, :]
    bt = bp.T

    ot, ost = pl.pallas_call(
        functools.partial(_nms_body, n_pad=n_pad, blk=blk),
        grid=(nb, nb),
        in_specs=[
            pl.BlockSpec((4, n_pad), lambda i, j: (0, 0)),
            pl.BlockSpec((blk, 4), lambda i, j: (i, 0)),
            pl.BlockSpec((1, n_pad), lambda i, j: (0, 0)),
        ],
        out_specs=[
            pl.BlockSpec((4, blk), lambda i, j: (0, i)),
            pl.BlockSpec((1, blk), lambda i, j: (0, i)),
        ],
        out_shape=[
            jax.ShapeDtypeStruct((4, n_pad), jnp.float32),
            jax.ShapeDtypeStruct((1, n_pad), jnp.float32),
        ],
        scratch_shapes=[
            pltpu.VMEM((blk, blk), jnp.float32),
            pltpu.VMEM((1, n_pad), jnp.float32),
        ],
    )(bt, bp, st)

    return jnp.concatenate([ot, ost], axis=0).T[:n]


# speedup vs baseline: 1.4834x; 1.4834x over previous
import functools

import jax
import jax.numpy as jnp
from jax import lax
from jax.experimental import pallas as pl
from jax.experimental.pallas import tpu as pltpu

_IOU_T = 0.5
_BLK = 512


def _nms_body(bt_ref, brow_ref, st_ref, ot_ref, os_ref,
              s_ref, keep_ref, *, n_pad, blk):
    i = pl.program_id(0)
    j = pl.program_id(1)
    ibase = pl.multiple_of(i * blk, blk)
    jbase = pl.multiple_of(j * blk, blk)

    @pl.when((i == 0) & (j == 0))
    def _():
        keep_ref[...] = jnp.ones_like(keep_ref)

    @pl.when(j >= i)
    def _active():
        x1 = brow_ref[:, 0:1]
        y1 = brow_ref[:, 1:2]
        x2 = brow_ref[:, 2:3]
        y2 = brow_ref[:, 3:4]
        cx1 = bt_ref[0:1, pl.ds(jbase, blk)]
        cy1 = bt_ref[1:2, pl.ds(jbase, blk)]
        cx2 = bt_ref[2:3, pl.ds(jbase, blk)]
        cy2 = bt_ref[3:4, pl.ds(jbase, blk)]
        area_r = (x2 - x1) * (y2 - y1)
        area_c = (cx2 - cx1) * (cy2 - cy1)
        iw = jnp.maximum(jnp.minimum(x2, cx2) - jnp.maximum(x1, cx1), 0.0)
        ih = jnp.maximum(jnp.minimum(y2, cy2) - jnp.maximum(y1, cy1), 0.0)
        inter = iw * ih
        union = jnp.maximum(area_r + area_c - inter, 1e-9)
        s = jnp.where(inter > _IOU_T * union, 1.0, 0.0)

        @pl.when(j == i)
        def _diagonal():
            tri_r = lax.broadcasted_iota(jnp.int32, (blk, blk), 0)
            tri_c = lax.broadcasted_iota(jnp.int32, (blk, blk), 1)
            s_ref[...] = jnp.where(tri_c > tri_r, s, 0.0)

            keep0 = keep_ref[:, pl.ds(ibase, blk)]

            def _cond(carry):
                return carry[1]

            def _sweep(carry):
                kb, _ = carry
                cnt = jnp.dot(kb, s_ref[...],
                              preferred_element_type=jnp.float32)
                kb_new = keep0 * jnp.where(cnt > 0.5, 0.0, 1.0)
                changed = jnp.sum(jnp.abs(kb_new - kb)) > 0.0
                return (kb_new, changed)

            kb, _ = lax.while_loop(_cond, _sweep, (keep0, True))
            keep_ref[:, pl.ds(ibase, blk)] = kb

            ot_ref[...] = bt_ref[:, pl.ds(ibase, blk)] * kb
            os_ref[...] = st_ref[:, pl.ds(ibase, blk)] * kb

        @pl.when(j > i)
        def _tail():
            kb = keep_ref[:, pl.ds(ibase, blk)]
            cnt = jnp.dot(kb, s, preferred_element_type=jnp.float32)
            keep_ref[:, pl.ds(jbase, blk)] = (
                keep_ref[:, pl.ds(jbase, blk)]
                * jnp.where(cnt > 0.5, 0.0, 1.0))


def kernel(boxes, scores):
    n = boxes.shape[0]
    blk = _BLK
    nb = -(-n // blk)
    n_pad = nb * blk

    order = jnp.argsort(-scores)
    b = jnp.take(boxes, order, axis=0)
    s = jnp.take(scores, order, axis=0)
    bp = jnp.concatenate(
        [b, jnp.zeros((n_pad - n, 4), jnp.float32)], axis=0)
    st = jnp.concatenate(
        [s, jnp.zeros((n_pad - n,), jnp.float32)], axis=0)[None, :]
    bt = bp.T

    ot, ost = pl.pallas_call(
        functools.partial(_nms_body, n_pad=n_pad, blk=blk),
        grid=(nb, nb),
        in_specs=[
            pl.BlockSpec((4, n_pad), lambda i, j: (0, 0)),
            pl.BlockSpec((blk, 4), lambda i, j: (i, 0)),
            pl.BlockSpec((1, n_pad), lambda i, j: (0, 0)),
        ],
        out_specs=[
            pl.BlockSpec((4, blk), lambda i, j: (0, i)),
            pl.BlockSpec((1, blk), lambda i, j: (0, i)),
        ],
        out_shape=[
            jax.ShapeDtypeStruct((4, n_pad), jnp.float32),
            jax.ShapeDtypeStruct((1, n_pad), jnp.float32),
        ],
        scratch_shapes=[
            pltpu.VMEM((blk, blk), jnp.float32),
            pltpu.VMEM((1, n_pad), jnp.float32),
        ],
    )(bt, bp, st)

    return jnp.concatenate([ot, ost], axis=0).T[:n]


# speedup vs baseline: 1.6483x; 1.1112x over previous
import functools

import jax
import jax.numpy as jnp
from jax import lax
from jax.experimental import pallas as pl
from jax.experimental.pallas import tpu as pltpu

_IOU_T = 0.5
_BLK = 1024


def _nms_body(bt_ref, brow_ref, st_ref, ot_ref, os_ref,
              s_ref, keep_ref, *, n_pad, blk):
    i = pl.program_id(0)
    j = pl.program_id(1)
    ibase = pl.multiple_of(i * blk, blk)
    jbase = pl.multiple_of(j * blk, blk)

    @pl.when((i == 0) & (j == 0))
    def _():
        keep_ref[...] = jnp.ones_like(keep_ref)

    @pl.when(j >= i)
    def _active():
        x1 = brow_ref[:, 0:1]
        y1 = brow_ref[:, 1:2]
        x2 = brow_ref[:, 2:3]
        y2 = brow_ref[:, 3:4]
        cx1 = bt_ref[0:1, pl.ds(jbase, blk)]
        cy1 = bt_ref[1:2, pl.ds(jbase, blk)]
        cx2 = bt_ref[2:3, pl.ds(jbase, blk)]
        cy2 = bt_ref[3:4, pl.ds(jbase, blk)]
        area_r = (x2 - x1) * (y2 - y1)
        area_c = (cx2 - cx1) * (cy2 - cy1)
        iw = jnp.maximum(jnp.minimum(x2, cx2) - jnp.maximum(x1, cx1), 0.0)
        ih = jnp.maximum(jnp.minimum(y2, cy2) - jnp.maximum(y1, cy1), 0.0)
        inter = iw * ih
        union = jnp.maximum(area_r + area_c - inter, 1e-9)
        s = jnp.where(inter > _IOU_T * union, 1.0, 0.0)

        @pl.when(j == i)
        def _diagonal():
            tri_r = lax.broadcasted_iota(jnp.int32, (blk, blk), 0)
            tri_c = lax.broadcasted_iota(jnp.int32, (blk, blk), 1)
            s_ref[...] = jnp.where(tri_c > tri_r, s, 0.0)

            keep0 = keep_ref[:, pl.ds(ibase, blk)]

            def _cond(carry):
                return carry[1]

            def _sweep(carry):
                kb, _ = carry
                cnt = jnp.dot(kb, s_ref[...],
                              preferred_element_type=jnp.float32)
                kb_new = keep0 * jnp.where(cnt > 0.5, 0.0, 1.0)
                changed = jnp.sum(jnp.abs(kb_new - kb)) > 0.0
                return (kb_new, changed)

            kb, _ = lax.while_loop(_cond, _sweep, (keep0, True))
            keep_ref[:, pl.ds(ibase, blk)] = kb

            ot_ref[...] = bt_ref[:, pl.ds(ibase, blk)] * kb
            os_ref[...] = st_ref[:, pl.ds(ibase, blk)] * kb

        @pl.when(j > i)
        def _tail():
            kb = keep_ref[:, pl.ds(ibase, blk)]
            cnt = jnp.dot(kb, s, preferred_element_type=jnp.float32)
            keep_ref[:, pl.ds(jbase, blk)] = (
                keep_ref[:, pl.ds(jbase, blk)]
                * jnp.where(cnt > 0.5, 0.0, 1.0))


def kernel(boxes, scores):
    n = boxes.shape[0]
    blk = _BLK
    nb = -(-n // blk)
    n_pad = nb * blk

    order = jnp.argsort(-scores)
    b = jnp.take(boxes, order, axis=0)
    s = jnp.take(scores, order, axis=0)
    bp = jnp.concatenate(
        [b, jnp.zeros((n_pad - n, 4), jnp.float32)], axis=0)
    st = jnp.concatenate(
        [s, jnp.zeros((n_pad - n,), jnp.float32)], axis=0)[None, :]
    bt = bp.T

    ot, ost = pl.pallas_call(
        functools.partial(_nms_body, n_pad=n_pad, blk=blk),
        grid=(nb, nb),
        in_specs=[
            pl.BlockSpec((4, n_pad), lambda i, j: (0, 0)),
            pl.BlockSpec((blk, 4), lambda i, j: (i, 0)),
            pl.BlockSpec((1, n_pad), lambda i, j: (0, 0)),
        ],
        out_specs=[
            pl.BlockSpec((4, blk), lambda i, j: (0, i)),
            pl.BlockSpec((1, blk), lambda i, j: (0, i)),
        ],
        out_shape=[
            jax.ShapeDtypeStruct((4, n_pad), jnp.float32),
            jax.ShapeDtypeStruct((1, n_pad), jnp.float32),
        ],
        scratch_shapes=[
            pltpu.VMEM((blk, blk), jnp.float32),
            pltpu.VMEM((1, n_pad), jnp.float32),
        ],
    )(bt, bp, st)

    return jnp.concatenate([ot, ost], axis=0).T[:n]
